# baseline (speedup 1.0000x reference)
import functools

import jax
import jax.numpy as jnp
from jax import lax
from jax.experimental import pallas as pl
from jax.experimental.pallas import tpu as pltpu
from jax.experimental.pallas import tpu_sc as plsc

BATCH = 16384
EMBED = 16
TABLE_ROWS = 1000000
NUM_CORES = 2
NUM_SUBCORES = 16
NUM_WORKERS = NUM_CORES * NUM_SUBCORES
SUBW = 1024
SHIFT = 10
NSUB_FULL = TABLE_ROWS // SUBW
LAST_SUB = NSUB_FULL
TAIL_ALIGNED = 512
TAIL_START = LAST_SUB * SUBW + TAIL_ALIGNED
KMAX = LAST_SUB // NUM_WORKERS + 1
IDXROWS = BATCH // 128
OUTW = 128


def _sc_gather(uidx2d, pidx2d, uT, pT, ub_flat, pb_flat, utail, ptail):
    mesh = plsc.VectorSubcoreMesh(core_axis_name="c", subcore_axis_name="s")

    @functools.partial(
        pl.kernel,
        mesh=mesh,
        compiler_params=pltpu.CompilerParams(
            use_tc_tiling_on_sc=True, needs_layout_passes=False),
        out_type=[
            jax.ShapeDtypeStruct((BATCH + 128, OUTW), jnp.float32),
            jax.ShapeDtypeStruct((BATCH + 128, OUTW), jnp.float32),
            jax.ShapeDtypeStruct((128, 128), jnp.float32),
        ],
        scratch_types=[
            pltpu.VMEM((IDXROWS, 128), jnp.int32),
            pltpu.VMEM((BATCH,), jnp.int32),
            pltpu.VMEM((2, EMBED, SUBW), jnp.float32),
            pltpu.VMEM((2, 128), jnp.int32),
            pltpu.VMEM((2, 128), jnp.int32),
            pltpu.VMEM((2, 128, OUTW), jnp.float32),
            pltpu.VMEM((NUM_WORKERS // 8, 128), jnp.float32),
            pltpu.VMEM((NUM_WORKERS // 8, 128), jnp.float32),
            pltpu.SemaphoreType.DMA,
            pltpu.SemaphoreType.DMA,
            pltpu.SemaphoreType.DMA,
            pltpu.SemaphoreType.DMA,
            pltpu.SemaphoreType.DMA,
        ],
    )
    def k(uidx_hbm, pidx_hbm, uT_hbm, pT_hbm, ub_hbm, pb_hbm,
          utail_hbm, ptail_hbm,
          urows_out, prows_out, bias_out,
          idx_v, mi_v, slab_v, iw_v, rw_v, stage_v,
          bu_v, bp_v, sem0, sem1, ssem0, ssem1, bsem):
        wid = lax.axis_index("s") * NUM_CORES + lax.axis_index("c")
        w16 = jnp.zeros((16,), jnp.int32) + wid
        iota = lax.iota(jnp.int32, 16)
        trash = jnp.zeros((16,), jnp.int32) + BATCH
        nbr = NUM_WORKERS // 8

        def fire_slab(table_hbm, tail_hbm, kk, sb, sem):
            s_id = wid + kk * NUM_WORKERS
            base = s_id * SUBW

            @pl.when(s_id < LAST_SUB)
            def _():
                pltpu.async_copy(
                    table_hbm.at[:, pl.ds(pl.multiple_of(base, 128), SUBW)],
                    slab_v.at[sb], sem)

            @pl.when(s_id == LAST_SUB)
            def _():
                pltpu.async_copy(
                    table_hbm.at[:, pl.ds(pl.multiple_of(base, 128),
                                          TAIL_ALIGNED)],
                    slab_v.at[sb, :, pl.ds(0, TAIL_ALIGNED)], sem)
                pltpu.async_copy(
                    tail_hbm, slab_v.at[sb, :, pl.ds(TAIL_ALIGNED, 128)], sem)

        def drain_slab(table_hbm, tail_hbm, kk, sb, sem):
            s_id = wid + kk * NUM_WORKERS
            base = s_id * SUBW

            @pl.when(s_id < LAST_SUB)
            def _():
                pltpu.make_async_copy(
                    table_hbm.at[:, pl.ds(pl.multiple_of(base, 128), SUBW)],
                    slab_v.at[sb], sem).wait()

            @pl.when(s_id == LAST_SUB)
            def _():
                pltpu.make_async_copy(
                    table_hbm.at[:, pl.ds(pl.multiple_of(base, 128),
                                          TAIL_ALIGNED)],
                    slab_v.at[sb, :, pl.ds(0, TAIL_ALIGNED)], sem).wait()
                pltpu.make_async_copy(
                    tail_hbm, slab_v.at[sb, :, pl.ds(TAIL_ALIGNED, 128)],
                    sem).wait()

        def table_pass(table_hbm, sidx_hbm, rows_out, bias_hbm, brows_v,
                       tail_hbm):
            pltpu.async_copy(sidx_hbm, idx_v, bsem).wait()

            for c in range(nbr):
                pltpu.async_copy(bias_hbm.at[idx_v.at[wid * nbr + c]],
                                 brows_v.at[c], bsem)

            fire_slab(table_hbm, tail_hbm, 0, 0, sem0)

            def scan_row(row, cnt):
                def scan_chunk(c, cnt):
                    r = idx_v[row, pl.ds(c * 16, 16)]
                    own = lax.shift_right_logical(r, SHIFT)
                    m = (own & (NUM_WORKERS - 1)) == w16
                    ivec = row * 128 + c * 16 + iota
                    plsc.store_compressed(mi_v.at[pl.ds(cnt, 16)], ivec,
                                          mask=m)
                    n = plsc.all_reduce_population_count(m)
                    return cnt + n[0]
                return lax.fori_loop(0, 8, scan_chunk, cnt)
            cnt = lax.fori_loop(0, IDXROWS, scan_row, jnp.int32(0))
            ngroups = lax.div(cnt + 127, jnp.int32(128))

            def extract_and_fire(sb, ssem, cnt2):
                def subwin(t, _):
                    cv = rw_v[sb, pl.ds(t * 16, 16)] & (SUBW - 1)
                    rows16 = t * 16 + iota
                    stg = stage_v.at[sb]
                    for j in range(EMBED):
                        jv = jnp.zeros((16,), jnp.int32) + j
                        vals = plsc.load_gather(slab_v.at[sb], [jv, cv])
                        plsc.store_scatter(stg, [rows16, jv], vals)
                    return 0
                nsub = lax.div(cnt2 + 15, jnp.int32(16))
                lax.fori_loop(0, nsub, subwin, 0)
                pltpu.async_copy(stage_v.at[sb], rows_out.at[iw_v.at[sb]],
                                 ssem)

            def wait_scatter(sb, ssem):
                pltpu.make_async_copy(stage_v.at[sb],
                                      rows_out.at[iw_v.at[sb]], ssem).wait()

            def trash_fill(sb):
                for t in range(8):
                    iw_v[sb, pl.ds(t * 16, 16)] = trash + t * 16 + iota

            def round_body(kk, sb, my_sem, other_sem, my_ssem, pend):
                s_id = wid + kk * NUM_WORKERS
                s16 = jnp.zeros((16,), jnp.int32) + s_id

                drain_slab(table_hbm, tail_hbm, kk, sb, my_sem)
                fire_slab(table_hbm, tail_hbm, kk + 1, 1 - sb, other_sem)

                @pl.when(pend > 0)
                def _():
                    wait_scatter(sb, my_ssem)
                trash_fill(sb)

                def group(g, cnt2):
                    def comp(c, cnt2):
                        @pl.when(cnt2 > 112)
                        def _():
                            extract_and_fire(sb, my_ssem, jnp.int32(128))
                            wait_scatter(sb, my_ssem)
                            trash_fill(sb)
                        cnt2 = jnp.where(cnt2 > 112, jnp.int32(0), cnt2)
                        pos = g * 128 + c * 16
                        i = mi_v[pl.ds(pos, 16)] & (BATCH - 1)
                        r = plsc.load_gather(
                            idx_v.at[:, :],
                            [lax.shift_right_logical(i, 7), i & 127])
                        valid = (pos + iota) < cnt
                        m2 = (lax.shift_right_logical(r, SHIFT) == s16) \
                            & valid
                        plsc.store_compressed(
                            rw_v.at[sb, pl.ds(cnt2, 16)], r, mask=m2)
                        plsc.store_compressed(
                            iw_v.at[sb, pl.ds(cnt2, 16)], i, mask=m2)
                        n = plsc.all_reduce_population_count(m2)
                        return cnt2 + n[0]
                    return lax.fori_loop(0, 8, comp, cnt2)

                cnt2 = lax.fori_loop(0, ngroups, group, jnp.int32(0))
                extract_and_fire(sb, my_ssem, cnt2)
                return jnp.int32(1)

            def subchunk_pair(t, carry):
                p0, p1 = carry
                p0 = round_body(2 * t, 0, sem0, sem1, ssem0, p0)
                p1 = round_body(2 * t + 1, 1, sem1, sem0, ssem1, p1)
                return (p0, p1)

            p0, p1 = lax.fori_loop(0, (KMAX + 1) // 2, subchunk_pair,
                                   (jnp.int32(0), jnp.int32(0)))

            @pl.when(p0 > 0)
            def _():
                wait_scatter(0, ssem0)

            @pl.when(p1 > 0)
            def _():
                wait_scatter(1, ssem1)

            for c in range(nbr):
                pltpu.make_async_copy(bias_hbm.at[idx_v.at[wid * nbr + c]],
                                      brows_v.at[c], bsem).wait()

        table_pass(uT_hbm, uidx_hbm, urows_out, ub_hbm, bu_v, utail_hbm)
        table_pass(pT_hbm, pidx_hbm, prows_out, pb_hbm, bp_v, ptail_hbm)

        for c in range(nbr):
            for t in range(8):
                sl = pl.ds(t * 16, 16)
                bu_v[c, sl] = bu_v[c, sl] + bp_v[c, sl]
        pltpu.sync_copy(bu_v, bias_out.at[pl.ds(wid * nbr, nbr)])

    return k(uidx2d, pidx2d, uT, pT, ub_flat, pb_flat, utail, ptail)


def _tc_finish(u_ref, p_ref, bias_ref, out_ref):
    u = u_ref[pl.ds(0, BATCH), pl.ds(0, EMBED)]
    p = p_ref[pl.ds(0, BATCH), pl.ds(0, EMBED)]
    s = jnp.sum(u * p)
    out_ref[...] = jax.nn.sigmoid(bias_ref[...] + s)


def kernel(inputs, user_embedding, user_bias, places_embedding, places_bias):
    uidx2d = inputs[:, 0].reshape(IDXROWS, 128)
    pidx2d = inputs[:, 1].reshape(IDXROWS, 128)
    utail = jnp.pad(user_embedding.T[:, TAIL_START:], ((0, 0), (0, 64)))
    ptail = jnp.pad(places_embedding.T[:, TAIL_START:], ((0, 0), (0, 64)))
    urows, prows, bias_sum = _sc_gather(
        uidx2d, pidx2d,
        user_embedding.T, places_embedding.T,
        user_bias.reshape(TABLE_ROWS), places_bias.reshape(TABLE_ROWS),
        utail, ptail)
    out2d = pl.pallas_call(
        _tc_finish,
        out_shape=jax.ShapeDtypeStruct((128, 128), jnp.float32),
    )(urows, prows, bias_sum)
    return out2d.reshape(BATCH, 1)
